# Initial kernel scaffold; baseline (speedup 1.0000x reference)
import jax
import jax.numpy as jnp
from jax.experimental import pallas as pl


def kernel(state, table):
    raise NotImplementedError("write your pallas kernel here")


# speedup vs baseline: 3.2252x; 3.2252x over previous
import jax
import jax.numpy as jnp
from jax import lax
from jax.experimental import pallas as pl
from jax.experimental.pallas import tpu as pltpu
from jax.experimental.pallas import tpu_sc as plsc

N = 16384
W = 200
D = 64
L = 16
NC, NS = 2, 16
NW = NC * NS
ROWS_PER_W = N // NW
CHUNK = 64
NCHUNK = ROWS_PER_W // CHUNK
WPAD = 208
G1 = 128
G2 = W - G1
NVD = D // L


def _gather_start(table_hbm, idx_ref, j, rows_ref, sem_a, sem_b):
    pltpu.make_async_copy(
        table_hbm.at[idx_ref.at[j, pl.ds(0, G1)]],
        rows_ref.at[pl.ds(0, G1)], sem_a).start()
    pltpu.make_async_copy(
        table_hbm.at[idx_ref.at[j, pl.ds(G1, G2)]],
        rows_ref.at[pl.ds(G1, G2)], sem_b).start()


def _gather_wait(table_hbm, idx_ref, j, rows_ref, sem_a, sem_b):
    pltpu.make_async_copy(
        table_hbm.at[idx_ref.at[j, pl.ds(0, G1)]],
        rows_ref.at[pl.ds(0, G1)], sem_a).wait()
    pltpu.make_async_copy(
        table_hbm.at[idx_ref.at[j, pl.ds(G1, G2)]],
        rows_ref.at[pl.ds(G1, G2)], sem_b).wait()


def _reduce_row(rows_ref, out_ref, r):
    def body(w, accs):
        return tuple(accs[d] + rows_ref[w, pl.ds(d * L, L)] for d in range(NVD))

    z = jnp.zeros((L,), jnp.float32)
    accs = lax.fori_loop(0, W, body, (z,) * NVD, unroll=4)
    scale = jnp.float32(1.0 / W)
    for d in range(NVD):
        out_ref[r, pl.ds(d * L, L)] = accs[d] * scale


def _sc_body(state_hbm, table_hbm, out_hbm, idx_buf, rows0, rows1, out_buf,
             sga0, sgb0, sga1, sgb1):
    wid = lax.axis_index("s") * NC + lax.axis_index("c")
    base = wid * ROWS_PER_W

    def chunk_body(c, _):
        row0 = base + c * CHUNK
        pltpu.sync_copy(state_hbm.at[pl.ds(row0, CHUNK), pl.ds(0, W)],
                        idx_buf.at[pl.ds(0, CHUNK), pl.ds(0, W)])

        def plus1(j, _):
            for v in range(WPAD // L):
                sl = pl.ds(v * L, L)
                idx_buf[j, sl] = idx_buf[j, sl] + 1
            return 0
        lax.fori_loop(0, CHUNK, plus1, 0)

        _gather_start(table_hbm, idx_buf, 0, rows0, sga0, sgb0)
        _gather_start(table_hbm, idx_buf, 1, rows1, sga1, sgb1)

        def pair(i, _):
            r0 = 2 * i
            _gather_wait(table_hbm, idx_buf, r0, rows0, sga0, sgb0)
            @pl.when(i < CHUNK // 2 - 1)
            def _():
                _gather_start(table_hbm, idx_buf, r0 + 2, rows0, sga0, sgb0)
            _reduce_row(rows0, out_buf, r0)

            r1 = r0 + 1
            _gather_wait(table_hbm, idx_buf, r1, rows1, sga1, sgb1)
            @pl.when(i < CHUNK // 2 - 1)
            def _():
                _gather_start(table_hbm, idx_buf, r1 + 2, rows1, sga1, sgb1)
            _reduce_row(rows1, out_buf, r1)
            return 0
        lax.fori_loop(0, CHUNK // 2, pair, 0)

        pltpu.sync_copy(out_buf, out_hbm.at[pl.ds(row0, CHUNK)])
        return 0

    lax.fori_loop(0, NCHUNK, chunk_body, 0)


def kernel(state, table):
    state2 = state.reshape(N, 2 * W).astype(jnp.int32)
    f = pl.kernel(
        _sc_body,
        out_type=jax.ShapeDtypeStruct((N, D), jnp.float32),
        mesh=plsc.VectorSubcoreMesh(core_axis_name="c", subcore_axis_name="s"),
        scratch_types=[
            pltpu.VMEM((CHUNK, WPAD), jnp.int32),
            pltpu.VMEM((W, D), jnp.float32),
            pltpu.VMEM((W, D), jnp.float32),
            pltpu.VMEM((CHUNK, D), jnp.float32),
            pltpu.SemaphoreType.DMA,
            pltpu.SemaphoreType.DMA,
            pltpu.SemaphoreType.DMA,
            pltpu.SemaphoreType.DMA,
        ],
        compiler_params=pltpu.CompilerParams(use_tc_tiling_on_sc=False),
    )
    return f(state2, table)


# speedup vs baseline: 3.3607x; 1.0420x over previous
import jax
import jax.numpy as jnp
from jax import lax
from jax.experimental import pallas as pl
from jax.experimental.pallas import tpu as pltpu
from jax.experimental.pallas import tpu_sc as plsc

N = 16384
W = 200
D = 64
L = 16
NC, NS = 2, 16
NW = NC * NS
ROWS_PER_W = N // NW
CHUNK = 64
NCHUNK = ROWS_PER_W // CHUNK
WPAD = 208
G1 = 128
G2 = W - G1
NVD = D // L


def _gather_start(table_hbm, idx_ref, j, rows_ref, sem_a, sem_b):
    pltpu.make_async_copy(
        table_hbm.at[idx_ref.at[j, pl.ds(0, G1)]],
        rows_ref.at[pl.ds(0, G1)], sem_a).start()
    pltpu.make_async_copy(
        table_hbm.at[idx_ref.at[j, pl.ds(G1, G2)]],
        rows_ref.at[pl.ds(G1, G2)], sem_b).start()


def _gather_wait(table_hbm, idx_ref, j, rows_ref, sem_a, sem_b):
    pltpu.make_async_copy(
        table_hbm.at[idx_ref.at[j, pl.ds(0, G1)]],
        rows_ref.at[pl.ds(0, G1)], sem_a).wait()
    pltpu.make_async_copy(
        table_hbm.at[idx_ref.at[j, pl.ds(G1, G2)]],
        rows_ref.at[pl.ds(G1, G2)], sem_b).wait()


def _reduce_row(rows_ref, out_ref, r):
    def body(w, accs):
        a = list(accs)
        for d in range(NVD):
            a[d] = a[d] + rows_ref[2 * w, pl.ds(d * L, L)]
        for d in range(NVD):
            a[NVD + d] = a[NVD + d] + rows_ref[2 * w + 1, pl.ds(d * L, L)]
        return tuple(a)

    z = jnp.zeros((L,), jnp.float32)
    accs = lax.fori_loop(0, W // 2, body, (z,) * (2 * NVD), unroll=4)
    scale = jnp.float32(1.0 / W)
    for d in range(NVD):
        out_ref[r, pl.ds(d * L, L)] = (accs[d] + accs[NVD + d]) * scale


def _sc_body(state_hbm, table_hbm, out_hbm, idx_buf, rows0, rows1, rows2,
             rows3, out_buf, sga0, sgb0, sga1, sgb1, sga2, sgb2, sga3, sgb3):
    wid = lax.axis_index("s") * NC + lax.axis_index("c")
    base = wid * ROWS_PER_W

    def chunk_body(c, _):
        row0 = base + c * CHUNK
        pltpu.sync_copy(state_hbm.at[pl.ds(row0, CHUNK), pl.ds(0, W)],
                        idx_buf.at[pl.ds(0, CHUNK), pl.ds(0, W)])

        def plus1(j, _):
            for v in range(WPAD // L):
                sl = pl.ds(v * L, L)
                idx_buf[j, sl] = idx_buf[j, sl] + 1
            return 0
        lax.fori_loop(0, CHUNK, plus1, 0)

        slots = ((rows0, sga0, sgb0), (rows1, sga1, sgb1),
                 (rows2, sga2, sgb2), (rows3, sga3, sgb3))
        NSLOT = len(slots)
        for k in range(NSLOT):
            _gather_start(table_hbm, idx_buf, k, *slots[k])

        def ring(i, _):
            for k in range(NSLOT):
                r = NSLOT * i + k
                _gather_wait(table_hbm, idx_buf, r, *slots[k])
                @pl.when(i < CHUNK // NSLOT - 1)
                def _():
                    _gather_start(table_hbm, idx_buf, r + NSLOT, *slots[k])
                _reduce_row(slots[k][0], out_buf, r)
            return 0
        lax.fori_loop(0, CHUNK // NSLOT, ring, 0)

        pltpu.sync_copy(out_buf, out_hbm.at[pl.ds(row0, CHUNK)])
        return 0

    lax.fori_loop(0, NCHUNK, chunk_body, 0)


def kernel(state, table):
    state2 = state.reshape(N, 2 * W).astype(jnp.int32)
    f = pl.kernel(
        _sc_body,
        out_type=jax.ShapeDtypeStruct((N, D), jnp.float32),
        mesh=plsc.VectorSubcoreMesh(core_axis_name="c", subcore_axis_name="s"),
        scratch_types=[
            pltpu.VMEM((CHUNK, WPAD), jnp.int32),
            pltpu.VMEM((W, D), jnp.float32),
            pltpu.VMEM((W, D), jnp.float32),
            pltpu.VMEM((W, D), jnp.float32),
            pltpu.VMEM((W, D), jnp.float32),
            pltpu.VMEM((CHUNK, D), jnp.float32),
        ] + [pltpu.SemaphoreType.DMA] * 8,
        compiler_params=pltpu.CompilerParams(use_tc_tiling_on_sc=False),
    )
    return f(state2, table)
